# Initial kernel scaffold; baseline (speedup 1.0000x reference)
import functools

import jax
import jax.numpy as jnp
from jax import lax
from jax.experimental import pallas as pl
from jax.experimental.pallas import tpu as pltpu
from jax.experimental.pallas import tpu_sc as plsc

N_NODES = 10000
N_FEAT = 128
N_HID = 128
N_CLASS = 64

NC, NS = 2, 16
NPAD = 10240
ROWS_PER_SUB = NPAD // NS
CHUNK = 128
N_CHUNKS_PER_SUB = 81
E_PER_SUB = CHUNK * N_CHUNKS_PER_SUB
E_PER_CORE = E_PER_SUB * NS
E_TOT = E_PER_CORE * NC

_mesh = plsc.VectorSubcoreMesh(core_axis_name="c", subcore_axis_name="s")


@functools.partial(
    pl.kernel,
    out_type=(jax.ShapeDtypeStruct((NPAD, 16), jnp.float32),
              jax.ShapeDtypeStruct((NPAD, 16), jnp.float32)),
    mesh=_mesh,
    scratch_types=[
        pltpu.VMEM((CHUNK,), jnp.int32),
        pltpu.VMEM((CHUNK, 16), jnp.float32),
        pltpu.VMEM_SHARED((NPAD, 16), jnp.float32),
    ],
)
def s_deg(dst_hbm, ones_hbm, zeros_hbm, out0, out1, dst_v, ones_v, acc):
    c = lax.axis_index("c")
    s = lax.axis_index("s")
    pltpu.sync_copy(zeros_hbm, acc.at[pl.ds(s * ROWS_PER_SUB, ROWS_PER_SUB)])
    pltpu.sync_copy(ones_hbm, ones_v)
    plsc.subcore_barrier()
    base0 = c * E_PER_CORE + s * E_PER_SUB

    def body(i, carry):
        base = base0 + i * CHUNK
        pltpu.sync_copy(dst_hbm.at[pl.ds(base, CHUNK)], dst_v)
        pltpu.sync_copy(ones_v, acc.at[dst_v], add=True)
        return carry

    lax.fori_loop(0, N_CHUNKS_PER_SUB, body, 0)
    plsc.subcore_barrier()
    rows = pl.ds(s * ROWS_PER_SUB, ROWS_PER_SUB)

    @pl.when(c == 0)
    def _():
        pltpu.sync_copy(acc.at[rows], out0.at[rows])

    @pl.when(c == 1)
    def _():
        pltpu.sync_copy(acc.at[rows], out1.at[rows])


@functools.partial(
    pl.kernel,
    out_type=(jax.ShapeDtypeStruct((NPAD, N_HID), jnp.float32),
              jax.ShapeDtypeStruct((NPAD, N_HID), jnp.float32)),
    mesh=_mesh,
    scratch_types=[
        pltpu.VMEM((CHUNK,), jnp.int32),
        pltpu.VMEM((CHUNK,), jnp.int32),
        pltpu.VMEM((CHUNK, N_HID), jnp.float32),
        pltpu.VMEM_SHARED((NPAD, N_HID), jnp.float32),
        pltpu.SemaphoreType.DMA,
    ],
)
def s_mp(h_hbm, src_hbm, dst_hbm, zeros_hbm, out0, out1,
         src_v, dst_v, rows_v, acc, sem):
    c = lax.axis_index("c")
    s = lax.axis_index("s")
    pltpu.sync_copy(zeros_hbm, acc.at[pl.ds(s * ROWS_PER_SUB, ROWS_PER_SUB)])
    plsc.subcore_barrier()
    base0 = c * E_PER_CORE + s * E_PER_SUB

    def body(i, carry):
        base = base0 + i * CHUNK
        pltpu.sync_copy(src_hbm.at[pl.ds(base, CHUNK)], src_v)
        pltpu.sync_copy(dst_hbm.at[pl.ds(base, CHUNK)], dst_v)
        pltpu.async_copy(h_hbm.at[src_v], rows_v, sem).wait()
        pltpu.sync_copy(rows_v, acc.at[dst_v], add=True)
        return carry

    lax.fori_loop(0, N_CHUNKS_PER_SUB, body, 0)
    plsc.subcore_barrier()
    rows = pl.ds(s * ROWS_PER_SUB, ROWS_PER_SUB)

    @pl.when(c == 0)
    def _():
        pltpu.sync_copy(acc.at[rows], out0.at[rows])

    @pl.when(c == 1)
    def _():
        pltpu.sync_copy(acc.at[rows], out1.at[rows])


ROW_BLK = 400
N_BLKS = N_NODES // ROW_BLK


def _dinv(d0, d1):
    return lax.rsqrt(jnp.maximum(d0[:, :1] + d1[:, :1], 1.0))


def _p1_body(x_ref, w_ref, d0_ref, d1_ref, o_ref):
    dinv = _dinv(d0_ref[...], d1_ref[...])
    o_ref[...] = jnp.dot(x_ref[...], w_ref[...],
                         preferred_element_type=jnp.float32) * dinv


def p1(x, w1, deg0, deg1):
    return pl.pallas_call(
        _p1_body,
        grid=(N_BLKS,),
        in_specs=[
            pl.BlockSpec((ROW_BLK, N_FEAT), lambda i: (i, 0)),
            pl.BlockSpec((N_FEAT, N_HID), lambda i: (0, 0)),
            pl.BlockSpec((ROW_BLK, 16), lambda i: (i, 0)),
            pl.BlockSpec((ROW_BLK, 16), lambda i: (i, 0)),
        ],
        out_specs=pl.BlockSpec((ROW_BLK, N_HID), lambda i: (i, 0)),
        out_shape=jax.ShapeDtypeStruct((N_NODES, N_HID), jnp.float32),
    )(x, w1, deg0, deg1)


def _p2_body(p0_ref, p1_ref, d0_ref, d1_ref, b_ref, w_ref, o_ref):
    dinv = _dinv(d0_ref[...], d1_ref[...])
    h = jnp.maximum((p0_ref[...] + p1_ref[...]) * dinv + b_ref[...], 0.0)
    o_ref[...] = jnp.dot(h, w_ref[...],
                         preferred_element_type=jnp.float32) * dinv


def p2(part0, part1, deg0, deg1, b1, w2):
    return pl.pallas_call(
        _p2_body,
        grid=(N_BLKS,),
        in_specs=[
            pl.BlockSpec((ROW_BLK, N_HID), lambda i: (i, 0)),
            pl.BlockSpec((ROW_BLK, N_HID), lambda i: (i, 0)),
            pl.BlockSpec((ROW_BLK, 16), lambda i: (i, 0)),
            pl.BlockSpec((ROW_BLK, 16), lambda i: (i, 0)),
            pl.BlockSpec((1, N_HID), lambda i: (0, 0)),
            pl.BlockSpec((N_HID, N_HID), lambda i: (0, 0)),
        ],
        out_specs=pl.BlockSpec((ROW_BLK, N_HID), lambda i: (i, 0)),
        out_shape=jax.ShapeDtypeStruct((N_NODES, N_HID), jnp.float32),
    )(part0, part1, deg0, deg1, b1, w2)


def _p3_body(p0_ref, p1_ref, d0_ref, d1_ref, b_ref, w_ref, b3_ref, o_ref):
    dinv = _dinv(d0_ref[...], d1_ref[...])
    h = jnp.maximum((p0_ref[...] + p1_ref[...]) * dinv + b_ref[...], 0.0)
    o_ref[...] = jnp.dot(h, w_ref[...],
                         preferred_element_type=jnp.float32) + b3_ref[...]


def p3(part0, part1, deg0, deg1, b2, w3, b3):
    return pl.pallas_call(
        _p3_body,
        grid=(N_BLKS,),
        in_specs=[
            pl.BlockSpec((ROW_BLK, N_HID), lambda i: (i, 0)),
            pl.BlockSpec((ROW_BLK, N_HID), lambda i: (i, 0)),
            pl.BlockSpec((ROW_BLK, 16), lambda i: (i, 0)),
            pl.BlockSpec((ROW_BLK, 16), lambda i: (i, 0)),
            pl.BlockSpec((1, N_HID), lambda i: (0, 0)),
            pl.BlockSpec((N_HID, N_CLASS), lambda i: (0, 0)),
            pl.BlockSpec((1, N_CLASS), lambda i: (0, 0)),
        ],
        out_specs=pl.BlockSpec((ROW_BLK, N_CLASS), lambda i: (i, 0)),
        out_shape=jax.ShapeDtypeStruct((N_NODES, N_CLASS), jnp.float32),
    )(part0, part1, deg0, deg1, b2, w3, b3)


def kernel(x, edge_index, W1, b1, W2, b2, W3, b3):
    ei = edge_index.astype(jnp.int32)
    loop = jnp.arange(N_NODES, dtype=jnp.int32)
    n_real = ei.shape[1] + N_NODES
    pad_n = E_TOT - n_real
    src_all = jnp.concatenate([ei[0], loop, jnp.zeros((pad_n,), jnp.int32)])
    dst_all = jnp.concatenate(
        [ei[1], loop, jnp.full((pad_n,), N_NODES, jnp.int32)])

    ones16 = jnp.ones((CHUNK, 16), jnp.float32)
    zeros16 = jnp.zeros((ROWS_PER_SUB, 16), jnp.float32)
    zerosH = jnp.zeros((ROWS_PER_SUB, N_HID), jnp.float32)

    deg0, deg1 = s_deg(dst_all, ones16, zeros16)
    hs0 = p1(x, W1, deg0[:N_NODES], deg1[:N_NODES])
    m0, m1 = s_mp(hs0, src_all, dst_all, zerosH)
    hs1 = p2(m0[:N_NODES], m1[:N_NODES], deg0[:N_NODES], deg1[:N_NODES],
             b1.reshape(1, N_HID), W2)
    n0, n1 = s_mp(hs1, src_all, dst_all, zerosH)
    out = p3(n0[:N_NODES], n1[:N_NODES], deg0[:N_NODES], deg1[:N_NODES],
             b2.reshape(1, N_HID), W3, b3.reshape(1, N_CLASS))
    return out


# speedup vs baseline: 12.2356x; 12.2356x over previous
import functools

import jax
import jax.numpy as jnp
from jax import lax
from jax.experimental import pallas as pl
from jax.experimental.pallas import tpu as pltpu
from jax.experimental.pallas import tpu_sc as plsc

N_NODES = 10000
N_FEAT = 128
N_HID = 128
N_CLASS = 64

NC, NS = 2, 16
NPAD = 10240
ROWS_PER_SUB = NPAD // NS
CHUNK = 128
N_CHUNKS_PER_SUB = 81
E_PER_SUB = CHUNK * N_CHUNKS_PER_SUB
E_PER_CORE = E_PER_SUB * NS
E_TOT = E_PER_CORE * NC

_mesh = plsc.VectorSubcoreMesh(core_axis_name="c", subcore_axis_name="s")


@functools.partial(
    pl.kernel,
    out_type=jax.ShapeDtypeStruct((NC * NPAD,), jnp.float32),
    mesh=_mesh,
    scratch_types=[
        pltpu.VMEM((CHUNK,), jnp.int32),
        pltpu.VMEM((CHUNK,), jnp.float32),
        pltpu.VMEM((ROWS_PER_SUB,), jnp.float32),
        pltpu.VMEM_SHARED((NPAD,), jnp.float32),
    ],
)
def s_deg(dst_hbm, out, dst_v, ones_v, zbuf, acc):
    c = lax.axis_index("c")
    s = lax.axis_index("s")
    ones = jnp.ones((16,), jnp.float32)
    zeros = jnp.zeros((16,), jnp.float32)

    def fill1(i, carry):
        ones_v[pl.ds(i * 16, 16)] = ones
        return carry

    lax.fori_loop(0, CHUNK // 16, fill1, 0)

    def fill0(i, carry):
        zbuf[pl.ds(i * 16, 16)] = zeros
        return carry

    lax.fori_loop(0, ROWS_PER_SUB // 16, fill0, 0)
    pltpu.sync_copy(zbuf, acc.at[pl.ds(s * ROWS_PER_SUB, ROWS_PER_SUB)])
    plsc.subcore_barrier()
    base0 = c * E_PER_CORE + s * E_PER_SUB

    def body(i, carry):
        base = base0 + i * CHUNK
        pltpu.sync_copy(dst_hbm.at[pl.ds(base, CHUNK)], dst_v)
        pltpu.sync_copy(ones_v, acc.at[dst_v], add=True)
        return carry

    lax.fori_loop(0, N_CHUNKS_PER_SUB, body, 0)
    plsc.subcore_barrier()
    rows = pl.ds(s * ROWS_PER_SUB, ROWS_PER_SUB)
    pltpu.sync_copy(acc.at[rows],
                    out.at[pl.ds(c * NPAD + s * ROWS_PER_SUB, ROWS_PER_SUB)])


@functools.partial(
    pl.kernel,
    out_type=jax.ShapeDtypeStruct((NC, NPAD, N_HID), jnp.float32),
    mesh=_mesh,
    scratch_types=[
        pltpu.VMEM((CHUNK,), jnp.int32),
        pltpu.VMEM((CHUNK,), jnp.int32),
        pltpu.VMEM((CHUNK, N_HID), jnp.float32),
        pltpu.VMEM_SHARED((NPAD, N_HID), jnp.float32),
        pltpu.SemaphoreType.DMA,
    ],
)
def s_mp(h_hbm, src_hbm, dst_hbm, zeros_hbm, out,
         src_v, dst_v, rows_v, acc, sem):
    c = lax.axis_index("c")
    s = lax.axis_index("s")
    pltpu.sync_copy(zeros_hbm, acc.at[pl.ds(s * ROWS_PER_SUB, ROWS_PER_SUB)])
    plsc.subcore_barrier()
    base0 = c * E_PER_CORE + s * E_PER_SUB

    def body(i, carry):
        base = base0 + i * CHUNK
        pltpu.sync_copy(src_hbm.at[pl.ds(base, CHUNK)], src_v)
        pltpu.sync_copy(dst_hbm.at[pl.ds(base, CHUNK)], dst_v)
        pltpu.async_copy(h_hbm.at[src_v], rows_v, sem).wait()
        pltpu.sync_copy(rows_v, acc.at[dst_v], add=True)
        return carry

    lax.fori_loop(0, N_CHUNKS_PER_SUB, body, 0)
    plsc.subcore_barrier()
    rows = pl.ds(s * ROWS_PER_SUB, ROWS_PER_SUB)
    pltpu.sync_copy(acc.at[rows], out.at[c, rows])


ROW_BLK = 400
N_BLKS = N_NODES // ROW_BLK


def _dinv(d0, d1):
    return lax.rsqrt(jnp.maximum(d0[:, :1] + d1[:, :1], 1.0))


def _p1_body(x_ref, w_ref, d0_ref, d1_ref, o_ref):
    dinv = _dinv(d0_ref[...], d1_ref[...])
    o_ref[...] = jnp.dot(x_ref[...], w_ref[...],
                         preferred_element_type=jnp.float32) * dinv


def p1(x, w1, deg0, deg1):
    return pl.pallas_call(
        _p1_body,
        grid=(N_BLKS,),
        in_specs=[
            pl.BlockSpec((ROW_BLK, N_FEAT), lambda i: (i, 0)),
            pl.BlockSpec((N_FEAT, N_HID), lambda i: (0, 0)),
            pl.BlockSpec((ROW_BLK, 1), lambda i: (i, 0)),
            pl.BlockSpec((ROW_BLK, 1), lambda i: (i, 0)),
        ],
        out_specs=pl.BlockSpec((ROW_BLK, N_HID), lambda i: (i, 0)),
        out_shape=jax.ShapeDtypeStruct((N_NODES, N_HID), jnp.float32),
    )(x, w1, deg0, deg1)


def _p2_body(p0_ref, p1_ref, d0_ref, d1_ref, b_ref, w_ref, o_ref):
    dinv = _dinv(d0_ref[...], d1_ref[...])
    h = jnp.maximum((p0_ref[...] + p1_ref[...]) * dinv + b_ref[...], 0.0)
    o_ref[...] = jnp.dot(h, w_ref[...],
                         preferred_element_type=jnp.float32) * dinv


def p2(part0, part1, deg0, deg1, b1, w2):
    return pl.pallas_call(
        _p2_body,
        grid=(N_BLKS,),
        in_specs=[
            pl.BlockSpec((ROW_BLK, N_HID), lambda i: (i, 0)),
            pl.BlockSpec((ROW_BLK, N_HID), lambda i: (i, 0)),
            pl.BlockSpec((ROW_BLK, 1), lambda i: (i, 0)),
            pl.BlockSpec((ROW_BLK, 1), lambda i: (i, 0)),
            pl.BlockSpec((1, N_HID), lambda i: (0, 0)),
            pl.BlockSpec((N_HID, N_HID), lambda i: (0, 0)),
        ],
        out_specs=pl.BlockSpec((ROW_BLK, N_HID), lambda i: (i, 0)),
        out_shape=jax.ShapeDtypeStruct((N_NODES, N_HID), jnp.float32),
    )(part0, part1, deg0, deg1, b1, w2)


def _p3_body(p0_ref, p1_ref, d0_ref, d1_ref, b_ref, w_ref, b3_ref, o_ref):
    dinv = _dinv(d0_ref[...], d1_ref[...])
    h = jnp.maximum((p0_ref[...] + p1_ref[...]) * dinv + b_ref[...], 0.0)
    o_ref[...] = jnp.dot(h, w_ref[...],
                         preferred_element_type=jnp.float32) + b3_ref[...]


def p3(part0, part1, deg0, deg1, b2, w3, b3):
    return pl.pallas_call(
        _p3_body,
        grid=(N_BLKS,),
        in_specs=[
            pl.BlockSpec((ROW_BLK, N_HID), lambda i: (i, 0)),
            pl.BlockSpec((ROW_BLK, N_HID), lambda i: (i, 0)),
            pl.BlockSpec((ROW_BLK, 1), lambda i: (i, 0)),
            pl.BlockSpec((ROW_BLK, 1), lambda i: (i, 0)),
            pl.BlockSpec((1, N_HID), lambda i: (0, 0)),
            pl.BlockSpec((N_HID, N_CLASS), lambda i: (0, 0)),
            pl.BlockSpec((1, N_CLASS), lambda i: (0, 0)),
        ],
        out_specs=pl.BlockSpec((ROW_BLK, N_CLASS), lambda i: (i, 0)),
        out_shape=jax.ShapeDtypeStruct((N_NODES, N_CLASS), jnp.float32),
    )(part0, part1, deg0, deg1, b2, w3, b3)


def kernel(x, edge_index, W1, b1, W2, b2, W3, b3):
    ei = edge_index.astype(jnp.int32)
    loop = jnp.arange(N_NODES, dtype=jnp.int32)
    n_real = ei.shape[1] + N_NODES
    pad_n = E_TOT - n_real
    src_all = jnp.concatenate([ei[0], loop, jnp.zeros((pad_n,), jnp.int32)])
    dst_all = jnp.concatenate(
        [ei[1], loop, jnp.full((pad_n,), N_NODES, jnp.int32)])

    zerosH = jnp.zeros((ROWS_PER_SUB, N_HID), jnp.float32)

    deg = s_deg(dst_all)
    deg0 = deg[:N_NODES].reshape(N_NODES, 1)
    deg1 = deg[NPAD:NPAD + N_NODES].reshape(N_NODES, 1)
    hs0 = p1(x, W1, deg0, deg1)
    m = s_mp(hs0, src_all, dst_all, zerosH)
    hs1 = p2(m[0, :N_NODES], m[1, :N_NODES], deg0, deg1,
             b1.reshape(1, N_HID), W2)
    n = s_mp(hs1, src_all, dst_all, zerosH)
    out = p3(n[0, :N_NODES], n[1, :N_NODES], deg0, deg1,
             b2.reshape(1, N_HID), W3, b3.reshape(1, N_CLASS))
    return out


# speedup vs baseline: 15.9506x; 1.3036x over previous
import functools

import jax
import jax.numpy as jnp
from jax import lax
from jax.experimental import pallas as pl
from jax.experimental.pallas import tpu as pltpu
from jax.experimental.pallas import tpu_sc as plsc

N_NODES = 10000
N_FEAT = 128
N_HID = 128
N_CLASS = 64

NC, NS = 2, 16
NPAD = 10240
ROWS_PER_SUB = NPAD // NS
CHUNK = 128
N_CHUNKS_PER_SUB = 81
E_PER_SUB = CHUNK * N_CHUNKS_PER_SUB
E_PER_CORE = E_PER_SUB * NS
E_TOT = E_PER_CORE * NC

_mesh = plsc.VectorSubcoreMesh(core_axis_name="c", subcore_axis_name="s")


@functools.partial(
    pl.kernel,
    out_type=jax.ShapeDtypeStruct((NC * NPAD,), jnp.float32),
    mesh=_mesh,
    scratch_types=[
        pltpu.VMEM((CHUNK,), jnp.int32),
        pltpu.VMEM((CHUNK,), jnp.float32),
        pltpu.VMEM((ROWS_PER_SUB,), jnp.float32),
        pltpu.VMEM_SHARED((NPAD,), jnp.float32),
    ],
)
def s_deg(dst_hbm, out, dst_v, ones_v, zbuf, acc):
    c = lax.axis_index("c")
    s = lax.axis_index("s")
    ones = jnp.ones((16,), jnp.float32)
    zeros = jnp.zeros((16,), jnp.float32)

    def fill1(i, carry):
        ones_v[pl.ds(i * 16, 16)] = ones
        return carry

    lax.fori_loop(0, CHUNK // 16, fill1, 0)

    def fill0(i, carry):
        zbuf[pl.ds(i * 16, 16)] = zeros
        return carry

    lax.fori_loop(0, ROWS_PER_SUB // 16, fill0, 0)
    pltpu.sync_copy(zbuf, acc.at[pl.ds(s * ROWS_PER_SUB, ROWS_PER_SUB)])
    plsc.subcore_barrier()
    base0 = c * E_PER_CORE + s * E_PER_SUB

    def body(i, carry):
        base = base0 + i * CHUNK
        pltpu.sync_copy(dst_hbm.at[pl.ds(base, CHUNK)], dst_v)
        pltpu.sync_copy(ones_v, acc.at[dst_v], add=True)
        return carry

    lax.fori_loop(0, N_CHUNKS_PER_SUB, body, 0)
    plsc.subcore_barrier()
    rows = pl.ds(s * ROWS_PER_SUB, ROWS_PER_SUB)
    pltpu.sync_copy(acc.at[rows],
                    out.at[pl.ds(c * NPAD + s * ROWS_PER_SUB, ROWS_PER_SUB)])


@functools.partial(
    pl.kernel,
    out_type=jax.ShapeDtypeStruct((NC, NPAD, N_HID), jnp.float32),
    mesh=_mesh,
    scratch_types=[
        pltpu.VMEM((CHUNK,), jnp.int32),
        pltpu.VMEM((CHUNK,), jnp.int32),
        pltpu.VMEM((CHUNK,), jnp.int32),
        pltpu.VMEM((CHUNK,), jnp.int32),
        pltpu.VMEM((CHUNK, N_HID), jnp.float32),
        pltpu.VMEM((CHUNK, N_HID), jnp.float32),
        pltpu.VMEM_SHARED((NPAD, N_HID), jnp.float32),
        pltpu.SemaphoreType.DMA,
        pltpu.SemaphoreType.DMA,
    ],
)
def s_mp(h_hbm, src_hbm, dst_hbm, zeros_hbm, out,
         src0, src1, dst0, dst1, rows0, rows1, acc, sem0, sem1):
    c = lax.axis_index("c")
    s = lax.axis_index("s")
    pltpu.sync_copy(zeros_hbm, acc.at[pl.ds(s * ROWS_PER_SUB, ROWS_PER_SUB)])
    plsc.subcore_barrier()
    base0 = c * E_PER_CORE + s * E_PER_SUB

    def fetch(chunk, src_v, dst_v, rows_v, sem):
        base = base0 + chunk * CHUNK
        pltpu.sync_copy(src_hbm.at[pl.ds(base, CHUNK)], src_v)
        pltpu.sync_copy(dst_hbm.at[pl.ds(base, CHUNK)], dst_v)
        pltpu.async_copy(h_hbm.at[src_v], rows_v, sem)

    def drain_scatter(src_v, dst_v, rows_v, sem):
        pltpu.make_async_copy(h_hbm.at[src_v], rows_v, sem).wait()
        pltpu.sync_copy(rows_v, acc.at[dst_v], add=True)

    fetch(0, src0, dst0, rows0, sem0)

    def body(k, carry):
        fetch(2 * k + 1, src1, dst1, rows1, sem1)
        drain_scatter(src0, dst0, rows0, sem0)
        fetch(2 * k + 2, src0, dst0, rows0, sem0)
        drain_scatter(src1, dst1, rows1, sem1)
        return carry

    lax.fori_loop(0, (N_CHUNKS_PER_SUB - 1) // 2, body, 0)
    drain_scatter(src0, dst0, rows0, sem0)

    plsc.subcore_barrier()
    rows = pl.ds(s * ROWS_PER_SUB, ROWS_PER_SUB)
    pltpu.sync_copy(acc.at[rows], out.at[c, rows])


ROW_BLK = 400
N_BLKS = N_NODES // ROW_BLK


def _dinv(d0, d1):
    return lax.rsqrt(jnp.maximum(d0[:, :1] + d1[:, :1], 1.0))


def _p1_body(x_ref, w_ref, d0_ref, d1_ref, o_ref):
    dinv = _dinv(d0_ref[...], d1_ref[...])
    o_ref[...] = jnp.dot(x_ref[...], w_ref[...],
                         preferred_element_type=jnp.float32) * dinv


def p1(x, w1, deg0, deg1):
    return pl.pallas_call(
        _p1_body,
        grid=(N_BLKS,),
        in_specs=[
            pl.BlockSpec((ROW_BLK, N_FEAT), lambda i: (i, 0)),
            pl.BlockSpec((N_FEAT, N_HID), lambda i: (0, 0)),
            pl.BlockSpec((ROW_BLK, 1), lambda i: (i, 0)),
            pl.BlockSpec((ROW_BLK, 1), lambda i: (i, 0)),
        ],
        out_specs=pl.BlockSpec((ROW_BLK, N_HID), lambda i: (i, 0)),
        out_shape=jax.ShapeDtypeStruct((N_NODES, N_HID), jnp.float32),
    )(x, w1, deg0, deg1)


def _p2_body(p0_ref, p1_ref, d0_ref, d1_ref, b_ref, w_ref, o_ref):
    dinv = _dinv(d0_ref[...], d1_ref[...])
    h = jnp.maximum((p0_ref[...] + p1_ref[...]) * dinv + b_ref[...], 0.0)
    o_ref[...] = jnp.dot(h, w_ref[...],
                         preferred_element_type=jnp.float32) * dinv


def p2(part0, part1, deg0, deg1, b1, w2):
    return pl.pallas_call(
        _p2_body,
        grid=(N_BLKS,),
        in_specs=[
            pl.BlockSpec((ROW_BLK, N_HID), lambda i: (i, 0)),
            pl.BlockSpec((ROW_BLK, N_HID), lambda i: (i, 0)),
            pl.BlockSpec((ROW_BLK, 1), lambda i: (i, 0)),
            pl.BlockSpec((ROW_BLK, 1), lambda i: (i, 0)),
            pl.BlockSpec((1, N_HID), lambda i: (0, 0)),
            pl.BlockSpec((N_HID, N_HID), lambda i: (0, 0)),
        ],
        out_specs=pl.BlockSpec((ROW_BLK, N_HID), lambda i: (i, 0)),
        out_shape=jax.ShapeDtypeStruct((N_NODES, N_HID), jnp.float32),
    )(part0, part1, deg0, deg1, b1, w2)


def _p3_body(p0_ref, p1_ref, d0_ref, d1_ref, b_ref, w_ref, b3_ref, o_ref):
    dinv = _dinv(d0_ref[...], d1_ref[...])
    h = jnp.maximum((p0_ref[...] + p1_ref[...]) * dinv + b_ref[...], 0.0)
    o_ref[...] = jnp.dot(h, w_ref[...],
                         preferred_element_type=jnp.float32) + b3_ref[...]


def p3(part0, part1, deg0, deg1, b2, w3, b3):
    return pl.pallas_call(
        _p3_body,
        grid=(N_BLKS,),
        in_specs=[
            pl.BlockSpec((ROW_BLK, N_HID), lambda i: (i, 0)),
            pl.BlockSpec((ROW_BLK, N_HID), lambda i: (i, 0)),
            pl.BlockSpec((ROW_BLK, 1), lambda i: (i, 0)),
            pl.BlockSpec((ROW_BLK, 1), lambda i: (i, 0)),
            pl.BlockSpec((1, N_HID), lambda i: (0, 0)),
            pl.BlockSpec((N_HID, N_CLASS), lambda i: (0, 0)),
            pl.BlockSpec((1, N_CLASS), lambda i: (0, 0)),
        ],
        out_specs=pl.BlockSpec((ROW_BLK, N_CLASS), lambda i: (i, 0)),
        out_shape=jax.ShapeDtypeStruct((N_NODES, N_CLASS), jnp.float32),
    )(part0, part1, deg0, deg1, b2, w3, b3)


def kernel(x, edge_index, W1, b1, W2, b2, W3, b3):
    ei = edge_index.astype(jnp.int32)
    loop = jnp.arange(N_NODES, dtype=jnp.int32)
    n_real = ei.shape[1] + N_NODES
    pad_n = E_TOT - n_real
    src_all = jnp.concatenate([ei[0], loop, jnp.zeros((pad_n,), jnp.int32)])
    dst_all = jnp.concatenate(
        [ei[1], loop, jnp.full((pad_n,), N_NODES, jnp.int32)])

    zerosH = jnp.zeros((ROWS_PER_SUB, N_HID), jnp.float32)

    deg = s_deg(dst_all)
    deg0 = deg[:N_NODES].reshape(N_NODES, 1)
    deg1 = deg[NPAD:NPAD + N_NODES].reshape(N_NODES, 1)
    hs0 = p1(x, W1, deg0, deg1)
    m = s_mp(hs0, src_all, dst_all, zerosH)
    hs1 = p2(m[0, :N_NODES], m[1, :N_NODES], deg0, deg1,
             b1.reshape(1, N_HID), W2)
    n = s_mp(hs1, src_all, dst_all, zerosH)
    out = p3(n[0, :N_NODES], n[1, :N_NODES], deg0, deg1,
             b2.reshape(1, N_HID), W3, b3.reshape(1, N_CLASS))
    return out
